# baseline (speedup 1.0000x reference)
import functools

import jax
import jax.numpy as jnp
from jax import lax
from jax.experimental import pallas as pl
from jax.experimental.pallas import tpu as pltpu
from jax.experimental.pallas import tpu_sc as plsc

EMBED = 64
LANES = 16
NUM_WORKERS = 32
HIST = 200
CHUNK = 400
NB = 4
SCALE = 8.0


PAIRS = CHUNK // 2


def _body(xe_hbm, xo_hbm, tab_hbm, out_hbm, *scratch):
    idxe = scratch[0:NB]
    idxo = scratch[NB:2 * NB]
    rows = scratch[2 * NB:3 * NB]
    gsem = scratch[3 * NB:4 * NB]
    ssem = scratch[4 * NB:5 * NB]

    wid = lax.axis_index("s") * 2 + lax.axis_index("c")
    n_pairs = xe_hbm.shape[0]
    per_w = n_pairs // NUM_WORKERS
    n_chunks = per_w // PAIRS
    base = wid * per_w

    def fill(c, b):
        start = base + c * PAIRS
        pltpu.sync_copy(xe_hbm.at[pl.ds(start, PAIRS)], idxe[b])
        pltpu.sync_copy(xo_hbm.at[pl.ds(start, PAIRS)], idxo[b])
        pltpu.async_copy(tab_hbm.at[idxe[b]], rows[b].at[0], gsem[b])
        pltpu.async_copy(tab_hbm.at[idxo[b]], rows[b].at[1], gsem[b])

    def wait_gather(b):
        for p in range(2):
            pltpu.make_async_copy(
                tab_hbm.at[pl.ds(0, PAIRS)], rows[b].at[p], gsem[b]
            ).wait()

    def store(c, b):
        start = base + c * PAIRS
        for p in range(2):
            pltpu.async_copy(
                rows[b].at[p],
                out_hbm.at[pl.ds(start, PAIRS), pl.ds(p * EMBED, EMBED)],
                ssem[b],
            )

    def wait_store(b):
        for p in range(2):
            pltpu.make_async_copy(
                rows[b].at[p],
                out_hbm.at[pl.ds(base, PAIRS), pl.ds(p * EMBED, EMBED)],
                ssem[b],
            ).wait()

    def scale(b):
        rb = rows[b]

        @plsc.parallel_loop(0, PAIRS, step=1, unroll=8)
        def _(r):
            for p in range(2):
                for j in range(EMBED // LANES):
                    sl = pl.ds(j * LANES, LANES)
                    rb[p, r, sl] = rb[p, r, sl] * SCALE

    fill(0, 0)
    fill(1, 1)

    def group(g, carry):
        for b in range(NB):
            c = g * NB + b
            br = (b + 2) % NB
            cr = c + 2

            @pl.when(cr < n_chunks)
            def _():
                @pl.when(c >= 2)
                def _():
                    wait_store(br)

                fill(cr, br)

            wait_gather(b)
            scale(b)
            store(c, b)
        return carry

    lax.fori_loop(0, n_chunks // NB, group, 0)
    for b in range(NB):
        wait_store(b)


def kernel(x, table):
    b, h = x.shape
    n = b * h
    xf = x.reshape(n).astype(jnp.int32)
    xe, xo = xf[0::2], xf[1::2]

    mesh = plsc.VectorSubcoreMesh(core_axis_name="c", subcore_axis_name="s")
    scratch = (
        [pltpu.VMEM((PAIRS,), jnp.int32) for _ in range(2 * NB)]
        + [pltpu.VMEM((2, PAIRS, EMBED), jnp.float32) for _ in range(NB)]
        + [pltpu.SemaphoreType.DMA for _ in range(2 * NB)]
    )
    k = functools.partial(
        pl.kernel,
        out_type=jax.ShapeDtypeStruct((n // 2, 2 * EMBED), jnp.float32),
        mesh=mesh,
        scratch_types=scratch,
        compiler_params=pltpu.CompilerParams(use_tc_tiling_on_sc=False),
    )(_body)
    out = k(xe, xo, table)
    return out.reshape(b, h, EMBED)


# speedup vs baseline: 1.3298x; 1.3298x over previous
import functools

import jax
import jax.numpy as jnp
from jax import lax
from jax.experimental import pallas as pl
from jax.experimental.pallas import tpu as pltpu
from jax.experimental.pallas import tpu_sc as plsc

EMBED = 64
LANES = 16
NUM_WORKERS = 32
HIST = 200
NB = 4
SCALE = 8.0


def _body(x_hbm, tab_hbm, out_hbm, *scratch):
    idx = scratch[0:NB]
    rows = scratch[NB:2 * NB]
    gsem = scratch[2 * NB:3 * NB]
    ssem = scratch[3 * NB:4 * NB]

    wid = lax.axis_index("s") * 2 + lax.axis_index("c")
    n_rows = x_hbm.shape[0] // HIST
    per_w = n_rows // NUM_WORKERS
    base = wid * per_w

    def fill(c, b):
        start = (base + c) * HIST
        pltpu.sync_copy(x_hbm.at[pl.ds(start, HIST)], idx[b])
        pltpu.async_copy(tab_hbm.at[idx[b]], rows[b], gsem[b])

    def wait_gather(b):
        pltpu.make_async_copy(
            tab_hbm.at[pl.ds(0, HIST)], rows[b], gsem[b]
        ).wait()

    def store(c, b):
        pltpu.async_copy(
            rows[b], out_hbm.at[base + c, :, pl.ds(0, EMBED)], ssem[b]
        )

    def wait_store(b):
        pltpu.make_async_copy(
            rows[b], out_hbm.at[base, :, pl.ds(0, EMBED)], ssem[b]
        ).wait()

    def scale(b):
        rb = rows[b]

        @plsc.parallel_loop(0, HIST, step=1, unroll=8)
        def _(r):
            for j in range(EMBED // LANES):
                sl = pl.ds(j * LANES, LANES)
                rb[r, sl] = rb[r, sl] * SCALE

    fill(0, 0)
    fill(1, 1)

    def group(g, carry):
        for b in range(NB):
            c = g * NB + b
            br = (b + 2) % NB
            cr = c + 2

            @pl.when(cr < per_w)
            def _():
                @pl.when(c >= 2)
                def _():
                    wait_store(br)

                fill(cr, br)

            wait_gather(b)
            scale(b)
            store(c, b)
        return carry

    lax.fori_loop(0, per_w // NB, group, 0)
    for b in range(NB):
        wait_store(b)


def kernel(x, table):
    b, h = x.shape
    n = b * h
    xf = x.reshape(n).astype(jnp.int32)

    mesh = plsc.VectorSubcoreMesh(core_axis_name="c", subcore_axis_name="s")
    scratch = (
        [pltpu.VMEM((HIST,), jnp.int32) for _ in range(NB)]
        + [pltpu.VMEM((HIST, EMBED), jnp.float32) for _ in range(NB)]
        + [pltpu.SemaphoreType.DMA for _ in range(2 * NB)]
    )
    k = functools.partial(
        pl.kernel,
        out_type=jax.ShapeDtypeStruct((b, h, 2 * EMBED), jnp.float32),
        mesh=mesh,
        scratch_types=scratch,
        compiler_params=pltpu.CompilerParams(use_tc_tiling_on_sc=False),
    )(_body)
    out = k(xf, table)
    return out[:, :, :EMBED]
